# baseline (speedup 1.0000x reference)
import functools

import jax
import jax.numpy as jnp
from jax import lax
from jax.experimental import pallas as pl
from jax.experimental.pallas import tpu as pltpu
from jax.experimental.pallas import tpu_sc as plsc

VOCAB = 1000000
DIM = 64
BATCH = 16384

_info = plsc.get_sparse_core_info()
_NC, _NS, _L = _info.num_cores, _info.num_subcores, _info.num_lanes
_NW = _NC * _NS
_BPW = BATCH // _NW
_CHUNK = 128
_NCH = _BPW // _CHUNK
_NGROUPS = _BPW // _L


def _sc_body(tidx_hbm, cidx_hbm, wt_hbm, wc_hbm, out_hbm,
             tidx_v, cidx_v, te_v, ce_v, dots_v, sem):
    wid = lax.axis_index("s") * _NC + lax.axis_index("c")
    base = wid * _BPW

    pltpu.sync_copy(tidx_hbm.at[wid], tidx_v)
    pltpu.sync_copy(cidx_hbm.at[wid], cidx_v)

    copies = []
    for j in range(_NCH):
        copies.append(pltpu.async_copy(
            wt_hbm.at[tidx_v.at[j]], te_v.at[pl.ds(j * _CHUNK, _CHUNK)], sem))
        copies.append(pltpu.async_copy(
            wc_hbm.at[cidx_v.at[j]], ce_v.at[pl.ds(j * _CHUNK, _CHUNK)], sem))
    for c in copies:
        c.wait()

    lane = lax.iota(jnp.int32, _L)

    def group_body(g, carry):
        rows = g * _L + lane
        acc = jnp.zeros((_L,), jnp.float32)
        for j in range(DIM):
            col = jnp.full((_L,), j, jnp.int32)
            tv = plsc.load_gather(te_v, [rows, col])
            cv = plsc.load_gather(ce_v, [rows, col])
            acc = acc + tv * cv
        dots_v[pl.ds(g * _L, _L)] = acc
        return carry

    lax.fori_loop(0, _NGROUPS, group_body, 0)

    pltpu.sync_copy(dots_v, out_hbm.at[pl.ds(base, _BPW)])


@jax.jit
def kernel(target, context, W_t, W_c):
    tidx = target.reshape(_NW, _NCH, _CHUNK).astype(jnp.int32)
    cidx = context.reshape(_NW, _NCH, _CHUNK).astype(jnp.int32)

    run = functools.partial(
        pl.kernel,
        out_type=jax.ShapeDtypeStruct((BATCH,), jnp.float32),
        mesh=plsc.VectorSubcoreMesh(core_axis_name="c", subcore_axis_name="s"),
        compiler_params=pltpu.CompilerParams(
            needs_layout_passes=False, use_tc_tiling_on_sc=False),
        scratch_types=[
            pltpu.VMEM((_NCH, _CHUNK), jnp.int32),
            pltpu.VMEM((_NCH, _CHUNK), jnp.int32),
            pltpu.VMEM((_BPW, DIM), jnp.float32),
            pltpu.VMEM((_BPW, DIM), jnp.float32),
            pltpu.VMEM((_BPW,), jnp.float32),
            pltpu.SemaphoreType.DMA,
        ],
    )(_sc_body)
    dots = run(tidx, cidx, W_t, W_c)
    return dots.reshape(BATCH, 1)


# speedup vs baseline: 1.2017x; 1.2017x over previous
import functools

import jax
import jax.numpy as jnp
from jax import lax
from jax.experimental import pallas as pl
from jax.experimental.pallas import tpu as pltpu
from jax.experimental.pallas import tpu_sc as plsc

VOCAB = 1000000
DIM = 64
BATCH = 16384

_info = plsc.get_sparse_core_info()
_NC, _NS, _L = _info.num_cores, _info.num_subcores, _info.num_lanes
_NW = _NC * _NS
_BPW = BATCH // _NW
_CHUNK = 128
_NCH = _BPW // _CHUNK
_GPC = _CHUNK // _L

_VBLK = 1024
_GRID = (VOCAB + _VBLK - 1) // _VBLK
_PROWS = _GRID * (_VBLK // 2)


def _tc_pack_body(wt_ref, wc_ref, pt_ref, pc_ref):
    for src, dst in ((wt_ref, pt_ref), (wc_ref, pc_ref)):
        x = src[...]
        dst[:, 0:DIM] = jnp.transpose(x[:, 0:_VBLK // 2])
        dst[:, DIM:2 * DIM] = jnp.transpose(x[:, _VBLK // 2:_VBLK])


def _pack_tables(wtT, wcT):
    return pl.pallas_call(
        _tc_pack_body,
        grid=(_GRID,),
        in_specs=[
            pl.BlockSpec((DIM, _VBLK), lambda i: (0, i)),
            pl.BlockSpec((DIM, _VBLK), lambda i: (0, i)),
        ],
        out_specs=[
            pl.BlockSpec((_VBLK // 2, 2 * DIM), lambda i: (i, 0)),
            pl.BlockSpec((_VBLK // 2, 2 * DIM), lambda i: (i, 0)),
        ],
        out_shape=[
            jax.ShapeDtypeStruct((_PROWS, 2 * DIM), jnp.float32),
            jax.ShapeDtypeStruct((_PROWS, 2 * DIM), jnp.float32),
        ],
    )(wtT, wcT)


def _sc_body(pit_hbm, pic_hbm, hot_hbm, hoc_hbm, pt_hbm, pc_hbm, out_hbm,
             pit_v, pic_v, hot_v, hoc_v, te0, te1, ce0, ce1, dots_v,
             semt, semc):
    wid = lax.axis_index("s") * _NC + lax.axis_index("c")
    base = wid * _BPW

    pltpu.sync_copy(pit_hbm.at[wid], pit_v)
    pltpu.sync_copy(pic_hbm.at[wid], pic_v)
    pltpu.sync_copy(hot_hbm.at[wid], hot_v)
    pltpu.sync_copy(hoc_hbm.at[wid], hoc_v)

    te_b = (te0, te1)
    ce_b = (ce0, ce1)
    lane = lax.iota(jnp.int32, _L)

    def fire(p):
        ht = pltpu.async_copy(pt_hbm.at[pit_v.at[p]], te_b[p % 2], semt)
        hc = pltpu.async_copy(pc_hbm.at[pic_v.at[p]], ce_b[p % 2], semc)
        return ht, hc

    def compute(p):
        te, ce = te_b[p % 2], ce_b[p % 2]

        def group_body(g, carry):
            gbase = p * _CHUNK + g * _L
            rows = g * _L + lane
            ht = hot_v[pl.ds(gbase, _L)]
            hc = hoc_v[pl.ds(gbase, _L)]
            acc = jnp.zeros((_L,), jnp.float32)
            for j in range(DIM):
                tv = plsc.load_gather(te, [rows, ht + j])
                cv = plsc.load_gather(ce, [rows, hc + j])
                acc = acc + tv * cv
            dots_v[pl.ds(gbase, _L)] = acc
            return carry

        lax.fori_loop(0, _GPC, group_body, 0)

    pending = fire(0)
    for p in range(_NCH):
        nxt = fire(p + 1) if p + 1 < _NCH else None
        pending[0].wait()
        pending[1].wait()
        compute(p)
        pending = nxt

    pltpu.sync_copy(dots_v, out_hbm.at[pl.ds(base, _BPW)])


@jax.jit
def kernel(target, context, W_t, W_c):
    pt, pc = _pack_tables(W_t.T, W_c.T)

    def prep(idx):
        v = idx.reshape(-1).astype(jnp.int32)
        p = (v >> 10) * (_VBLK // 2) + (v & (_VBLK // 2 - 1))
        hoff = ((v >> 9) & 1) * DIM
        return p.reshape(_NW, _NCH, _CHUNK), hoff.reshape(_NW, _BPW)

    pit, hot = prep(target)
    pic, hoc = prep(context)

    run = functools.partial(
        pl.kernel,
        out_type=jax.ShapeDtypeStruct((BATCH,), jnp.float32),
        mesh=plsc.VectorSubcoreMesh(core_axis_name="c", subcore_axis_name="s"),
        compiler_params=pltpu.CompilerParams(
            needs_layout_passes=False, use_tc_tiling_on_sc=True),
        scratch_types=[
            pltpu.VMEM((_NCH, _CHUNK), jnp.int32),
            pltpu.VMEM((_NCH, _CHUNK), jnp.int32),
            pltpu.VMEM((_BPW,), jnp.int32),
            pltpu.VMEM((_BPW,), jnp.int32),
            pltpu.VMEM((_CHUNK, 2 * DIM), jnp.float32),
            pltpu.VMEM((_CHUNK, 2 * DIM), jnp.float32),
            pltpu.VMEM((_CHUNK, 2 * DIM), jnp.float32),
            pltpu.VMEM((_CHUNK, 2 * DIM), jnp.float32),
            pltpu.VMEM((_BPW,), jnp.float32),
            pltpu.SemaphoreType.DMA,
            pltpu.SemaphoreType.DMA,
        ],
    )(_sc_body)
    dots = run(pit, pic, hot, hoc, pt, pc)
    return dots.reshape(BATCH, 1)
